# baseline (speedup 1.0000x reference)
import jax
import jax.numpy as jnp
from jax.experimental import pallas as pl
from jax.experimental.pallas import tpu as pltpu

H = 4096
E = 64
N = 8192
TOP_K = 8

_PREC = jax.lax.Precision.DEFAULT

BLK = 512
N_STREAM = N // BLK
CE1_BLK = 256
N_CE1 = (H // 2) // CE1_BLK
RA1_BLK = H // 2
N_RA1 = H // RA1_BLK
EB = 512
N_EPI = N // EB

CTX_STEP = N_STREAM + N_CE1
RA1_0 = CTX_STEP + 1
ADAPT_STEP = RA1_0 + N_RA1
EPI0 = ADAPT_STEP + 1
GRID = EPI0 + N_EPI


def _fused_kernel(h_ref, wb_ref, wc1_ref, bc1_ref, wc2_ref, bc2_ref,
                  wr1_ref, br1_ref, wr2_ref, br2_ref,
                  out_ref, idx_ref, wts_ref, var_ref, ent_ref,
                  logits_scr, colsum_scr, h1_scr, ctx_scr, a1_scr,
                  adapt_scr, load_acc, ent_acc):
    i = pl.program_id(0)

    @pl.when(i < N_STREAM)
    def _():
        h = h_ref[...]
        logits_scr[i] = jax.lax.dot_general(
            h, wb_ref[...], (((1,), (1,)), ((), ())),
            preferred_element_type=jnp.float32, precision=_PREC).T
        part = jnp.sum(h, axis=0, keepdims=True)

        @pl.when(i == 0)
        def _():
            colsum_scr[...] = part

        @pl.when(i != 0)
        def _():
            colsum_scr[...] += part

    @pl.when(jnp.logical_and(i >= N_STREAM, i < CTX_STEP))
    def _():
        pooled = colsum_scr[...] * (1.0 / N)
        v = jnp.maximum(jax.lax.dot_general(
            pooled, wc1_ref[...], (((1,), (1,)), ((), ())),
            preferred_element_type=jnp.float32, precision=_PREC)
            + bc1_ref[...], 0.0)
        for s in range(N_CE1):
            @pl.when(i == N_STREAM + s)
            def _():
                half = (s % 2) * CE1_BLK
                h1_scr[s // 2:s // 2 + 1, half:half + CE1_BLK] = v

    @pl.when(i == CTX_STEP)
    def _():
        context = bc2_ref[...]
        for s in range(4):
            context += jax.lax.dot_general(
                h1_scr[s:s + 1, :],
                wc2_ref[:, s * 512:(s + 1) * 512],
                (((1,), (1,)), ((), ())),
                preferred_element_type=jnp.float32, precision=_PREC)
        ctx_scr[...] = context

    @pl.when(jnp.logical_and(i >= RA1_0, i < ADAPT_STEP))
    def _():
        v = jnp.tanh(jax.lax.dot_general(
            ctx_scr[...], wr1_ref[...], (((1,), (1,)), ((), ())),
            preferred_element_type=jnp.float32, precision=_PREC)
            + br1_ref[...])
        for t in range(N_RA1):
            @pl.when(i == RA1_0 + t)
            def _():
                a1_scr[:, t * RA1_BLK:(t + 1) * RA1_BLK] = v

    @pl.when(i == ADAPT_STEP)
    def _():
        adapt = jax.lax.dot_general(
            a1_scr[...], wr2_ref[...], (((1,), (1,)), ((), ())),
            preferred_element_type=jnp.float32, precision=_PREC) + br2_ref[...]
        adapt_scr[...] = adapt.reshape(E, 1)

    @pl.when(i >= EPI0)
    def _():
        j = i - EPI0
        x = logits_scr[j] + adapt_scr[...]
        out_ref[...] = x.T

        m = jnp.max(x, axis=0, keepdims=True)
        ex = jnp.exp(x - m)
        s = jnp.sum(ex, axis=0, keepdims=True)
        probs = ex / s
        row_ent = -jnp.sum(probs * jnp.log(probs + 1e-8), axis=0)
        ent_part = jnp.sum(row_ent).reshape(1, 1)
        load_part = jnp.sum(probs, axis=1, keepdims=True)

        @pl.when(i == EPI0)
        def _():
            load_acc[...] = load_part
            ent_acc[...] = ent_part

        @pl.when(i != EPI0)
        def _():
            load_acc[...] += load_part
            ent_acc[...] += ent_part

        ii = jax.lax.broadcasted_iota(jnp.int32, x.shape, 0)
        vals = []
        idxs = []
        for _k in range(TOP_K):
            mval = jnp.max(x, axis=0, keepdims=True)
            cand = jnp.where(x == mval, ii, E)
            am = jnp.min(cand, axis=0, keepdims=True)
            vals.append(mval)
            idxs.append(am)
            x = jnp.where(ii == am, -jnp.inf, x)
        topv = jnp.concatenate(vals, axis=0)
        topi = jnp.concatenate(idxs, axis=0)
        idx_ref[...] = topi.T
        e2 = jnp.exp(topv - topv[:1, :])
        wts_ref[...] = (e2 / jnp.sum(e2, axis=0, keepdims=True)).T

        @pl.when(i == EPI0 + N_EPI - 1)
        def _():
            el = load_acc[...] * (1.0 / N)
            mu = jnp.mean(el)
            var_ref[...] = (jnp.sum((el - mu) ** 2)
                            * (1.0 / (E - 1))).reshape(1, 1)
            ent_ref[...] = ent_acc[...] * (1.0 / N)


def kernel(hidden_states, W_base, W_ce1, b_ce1, W_ce2, b_ce2, W_ra1, b_ra1,
           W_ra2, b_ra2):
    H2 = H // 2
    H4 = H // 4

    def h_map(i):
        return (jnp.minimum(i, N_STREAM - 1), 0)

    def ce1_map(i):
        return (jnp.clip(i - N_STREAM, 0, N_CE1 - 1), 0)

    def bce1_map(i):
        return (0, jnp.clip(i - N_STREAM, 0, N_CE1 - 1))

    def ra1_map(i):
        return (jnp.clip(i - RA1_0, 0, N_RA1 - 1), 0)

    def bra1_map(i):
        return (0, jnp.clip(i - RA1_0, 0, N_RA1 - 1))

    def epi_map_t(i):
        return (jnp.clip(i - EPI0, 0, N_EPI - 1), 0)

    const2 = lambda i: (0, 0)

    adapted, idx, wts, var_out, ent_out = pl.pallas_call(
        _fused_kernel,
        grid=(GRID,),
        in_specs=[
            pl.BlockSpec((BLK, H), h_map),
            pl.BlockSpec((E, H), const2),
            pl.BlockSpec((CE1_BLK, H), ce1_map),
            pl.BlockSpec((1, CE1_BLK), bce1_map),
            pl.BlockSpec((H4, H2), const2),
            pl.BlockSpec((1, H4), const2),
            pl.BlockSpec((RA1_BLK, H4), ra1_map),
            pl.BlockSpec((1, RA1_BLK), bra1_map),
            pl.BlockSpec((E, H), const2),
            pl.BlockSpec((1, E), const2),
        ],
        out_specs=[
            pl.BlockSpec((EB, E), epi_map_t),
            pl.BlockSpec((EB, TOP_K), epi_map_t),
            pl.BlockSpec((EB, TOP_K), epi_map_t),
            pl.BlockSpec((1, 1), const2),
            pl.BlockSpec((1, 1), const2),
        ],
        out_shape=[
            jax.ShapeDtypeStruct((N, E), jnp.float32),
            jax.ShapeDtypeStruct((N, TOP_K), jnp.int32),
            jax.ShapeDtypeStruct((N, TOP_K), jnp.float32),
            jax.ShapeDtypeStruct((1, 1), jnp.float32),
            jax.ShapeDtypeStruct((1, 1), jnp.float32),
        ],
        scratch_shapes=[
            pltpu.VMEM((N_STREAM, E, BLK), jnp.float32),
            pltpu.VMEM((1, H), jnp.float32),
            pltpu.VMEM((4, 512), jnp.float32),
            pltpu.VMEM((1, H4), jnp.float32),
            pltpu.VMEM((1, H), jnp.float32),
            pltpu.VMEM((E, 1), jnp.float32),
            pltpu.VMEM((E, 1), jnp.float32),
            pltpu.VMEM((1, 1), jnp.float32),
        ],
    )(hidden_states, W_base, W_ce1, b_ce1.reshape(1, H2), W_ce2,
      b_ce2.reshape(1, H4), W_ra1, b_ra1.reshape(1, H), W_ra2,
      b_ra2.reshape(1, E))

    return (adapted, idx, wts, var_out[0, 0], ent_out[0, 0])
